# baseline (speedup 1.0000x reference)
import jax
import jax.numpy as jnp
from jax.experimental import pallas as pl
from jax.experimental.pallas import tpu as pltpu


def _expand_kernel(x_ref, w_ref, b_ref, o_ref):
    acc = jnp.dot(x_ref[...], w_ref[...], preferred_element_type=jnp.float32)
    o_ref[...] = (acc + b_ref[...].astype(jnp.float32)).astype(o_ref.dtype)


def kernel(x, weight, bias):
    B, Cin = x.shape
    F = weight.shape[1]
    out_channels, Y, X = 128, 8, 8

    x_bf = x.astype(jnp.bfloat16)
    w_bf = weight.astype(jnp.bfloat16)

    TN = next((t for t in (512, 256, 128) if F % t == 0), F)
    num_j = F // TN

    out_flat = pl.pallas_call(
        _expand_kernel,
        out_shape=jax.ShapeDtypeStruct((B, F), x.dtype),
        grid=(num_j,),
        in_specs=[
            pl.BlockSpec((B, Cin), lambda j: (0, 0)),
            pl.BlockSpec((Cin, TN), lambda j: (0, j)),
            pl.BlockSpec((1, TN), lambda j: (0, j)),
        ],
        out_specs=pl.BlockSpec((B, TN), lambda j: (0, j)),
        compiler_params=pltpu.CompilerParams(
            dimension_semantics=("parallel",)),
        cost_estimate=pl.CostEstimate(
            flops=2 * B * Cin * F,
            transcendentals=0,
            bytes_accessed=(B * Cin + Cin * F) * 2 + B * F * 4,
        ),
    )(x_bf, w_bf, bias.reshape(1, F))

    return out_flat.reshape(B, out_channels, Y, X)


# speedup vs baseline: 2.2532x; 2.2532x over previous
import jax
import jax.numpy as jnp
from jax.experimental import pallas as pl
from jax.experimental.pallas import tpu as pltpu


def _prep_kernel(w_ref, x_ref, wp_ref, xb_ref):
    v = w_ref[...].reshape(w_ref.shape[0], 128, 64)
    wp_ref[...] = jnp.swapaxes(v, 1, 2).astype(wp_ref.dtype).reshape(wp_ref.shape)
    xb_ref[...] = x_ref[...].astype(xb_ref.dtype)


def _expand_kernel(x_ref, w_ref, b_ref, o_ref):
    acc = jnp.dot(x_ref[...], w_ref[...], preferred_element_type=jnp.float32)
    acc = acc + b_ref[...]
    o_ref[...] = acc.astype(o_ref.dtype).reshape(o_ref.shape)


def kernel(x, weight, bias):
    B, Cin = x.shape
    F = weight.shape[1]
    C, Y, X = 128, 8, 8
    S = Y * X

    b_perm = bias.reshape(C, Y, X).transpose(1, 2, 0).reshape(1, F)

    TK = Cin // 8
    TB = B // 8
    w_perm, x_bf = pl.pallas_call(
        _prep_kernel,
        out_shape=(jax.ShapeDtypeStruct((Cin, F), jnp.bfloat16),
                   jax.ShapeDtypeStruct((B, Cin), jnp.bfloat16)),
        grid=(8,),
        in_specs=[
            pl.BlockSpec((TK, F), lambda j: (j, 0)),
            pl.BlockSpec((TB, Cin), lambda j: (j, 0)),
        ],
        out_specs=(pl.BlockSpec((TK, F), lambda j: (j, 0)),
                   pl.BlockSpec((TB, Cin), lambda j: (j, 0))),
        compiler_params=pltpu.CompilerParams(
            dimension_semantics=("parallel",)),
    )(weight, x)

    TM = 128
    num_m = B // (2 * TM)

    out4 = pl.pallas_call(
        _expand_kernel,
        out_shape=jax.ShapeDtypeStruct((B // 8, 8, S, C), x.dtype),
        grid=(2, num_m),
        in_specs=[
            pl.BlockSpec((TM, Cin), lambda c, m: (c * num_m + m, 0)),
            pl.BlockSpec((Cin, F), lambda c, m: (0, 0)),
            pl.BlockSpec((1, F), lambda c, m: (0, 0)),
        ],
        out_specs=pl.BlockSpec((TM // 8, 8, S, C),
                               lambda c, m: (c * num_m + m, 0, 0, 0)),
        compiler_params=pltpu.CompilerParams(
            dimension_semantics=("parallel", "arbitrary")),
        cost_estimate=pl.CostEstimate(
            flops=2 * B * Cin * F,
            transcendentals=0,
            bytes_accessed=(B * Cin + Cin * F) * 2 + B * F * 4,
        ),
    )(x_bf, w_perm, b_perm)

    return out4.reshape(B, Y, X, C).transpose(0, 3, 1, 2)


# speedup vs baseline: 2.3022x; 1.0217x over previous
import jax
import jax.numpy as jnp
from jax.experimental import pallas as pl
from jax.experimental.pallas import tpu as pltpu


def _prep_kernel(w_ref, wp_ref):
    v = w_ref[...].reshape(w_ref.shape[0], 128, 64)
    wp_ref[...] = jnp.swapaxes(v, 1, 2).astype(wp_ref.dtype).reshape(wp_ref.shape)


def _expand_kernel(x_ref, w_ref, b_ref, o_ref):
    xb = x_ref[...].astype(jnp.bfloat16)
    acc = jnp.dot(xb, w_ref[...], preferred_element_type=jnp.float32)
    acc = acc + b_ref[...]
    o_ref[...] = acc.astype(o_ref.dtype).reshape(o_ref.shape)


def kernel(x, weight, bias):
    B, Cin = x.shape
    F = weight.shape[1]
    C, Y, X = 128, 8, 8
    S = Y * X

    b_perm = bias.reshape(C, Y, X).transpose(1, 2, 0).reshape(1, F)

    TK = Cin // 8
    w_perm = pl.pallas_call(
        _prep_kernel,
        out_shape=jax.ShapeDtypeStruct((Cin, F), jnp.bfloat16),
        grid=(8,),
        in_specs=[pl.BlockSpec((TK, F), lambda j: (j, 0))],
        out_specs=pl.BlockSpec((TK, F), lambda j: (j, 0)),
        compiler_params=pltpu.CompilerParams(
            dimension_semantics=("parallel",)),
    )(weight)

    TM = 128
    num_m = B // (2 * TM)

    out4 = pl.pallas_call(
        _expand_kernel,
        out_shape=jax.ShapeDtypeStruct((B // 8, 8, S, C), x.dtype),
        grid=(2, num_m),
        in_specs=[
            pl.BlockSpec((TM, Cin), lambda c, m: (c * num_m + m, 0)),
            pl.BlockSpec((Cin, F), lambda c, m: (0, 0)),
            pl.BlockSpec((1, F), lambda c, m: (0, 0)),
        ],
        out_specs=pl.BlockSpec((TM // 8, 8, S, C),
                               lambda c, m: (c * num_m + m, 0, 0, 0)),
        compiler_params=pltpu.CompilerParams(
            dimension_semantics=("parallel", "arbitrary")),
        cost_estimate=pl.CostEstimate(
            flops=2 * B * Cin * F,
            transcendentals=0,
            bytes_accessed=(B * Cin + Cin * F) * 2 + B * F * 4,
        ),
    )(x, w_perm, b_perm)

    return out4.reshape(B, Y, X, C).transpose(0, 3, 1, 2)
